# baseline (speedup 1.0000x reference)
import functools

import jax
import jax.numpy as jnp
from jax import lax
from jax.experimental import pallas as pl
from jax.experimental.pallas import tpu as pltpu
from jax.experimental.pallas import tpu_sc as plsc

_BR = 1024


def _vq_block(z_ref, e_ref, esq_ref, idx_ref, loss_ref):
    i = pl.program_id(0)
    z = z_ref[...]
    e = e_ref[...]
    z_sq = jnp.sum(jnp.square(z), axis=1, keepdims=True)
    e_sq = esq_ref[0, :]
    m2_z_dot_e = jax.lax.dot_general(
        z * -2.0, e, (((1,), (1,)), ((), ())),
        preferred_element_type=jnp.float32)
    dist = (z_sq + m2_z_dot_e) + e_sq[None, :]
    idx = jnp.argmin(dist, axis=1).astype(jnp.int32)
    idx_ref[0, 0, :] = idx
    part = jnp.sum(jnp.min(dist, axis=1))

    @pl.when(i == 0)
    def _init():
        loss_ref[0, 0] = part

    @pl.when(i != 0)
    def _acc():
        loss_ref[0, 0] += part


def _sc_gather_call(table, idx_flat, m, d):
    info = plsc.get_sparse_core_info()
    nw = info.num_cores * info.num_subcores
    b_per_w = m // nw
    chunk = 96
    nchunk = b_per_w // chunk
    mesh = plsc.VectorSubcoreMesh(core_axis_name="c", subcore_axis_name="s")

    @functools.partial(
        pl.kernel, mesh=mesh,
        compiler_params=pltpu.CompilerParams(use_tc_tiling_on_sc=False),
        out_type=jax.ShapeDtypeStruct((m, d), jnp.float32),
        scratch_types=[
            pltpu.VMEM((b_per_w,), jnp.int32),
            pltpu.VMEM((b_per_w, d), jnp.float32),
            pltpu.SemaphoreType.DMA,
        ],
    )
    def gather(table_hbm, idx_hbm, out_hbm, idx_v, rows_v, sem):
        wid = lax.axis_index("s") * info.num_cores + lax.axis_index("c")
        base = wid * b_per_w
        pltpu.sync_copy(idx_hbm.at[pl.ds(base, b_per_w)], idx_v)
        copies = [
            pltpu.async_copy(
                table_hbm.at[idx_v.at[pl.ds(j * chunk, chunk)]],
                rows_v.at[pl.ds(j * chunk, chunk)], sem)
            for j in range(nchunk)
        ]
        for c in copies:
            c.wait()
        pltpu.sync_copy(rows_v, out_hbm.at[pl.ds(base, b_per_w)])

    return gather(table, idx_flat)


def _vq_tc_call(z_part, embedding_weight, e_sq_row, k, d):
    m = z_part.shape[0]
    nblk = m // _BR
    return pl.pallas_call(
        _vq_block,
        grid=(nblk,),
        in_specs=[
            pl.BlockSpec((_BR, d), lambda i: (i, 0)),
            pl.BlockSpec((k, d), lambda i: (0, 0)),
            pl.BlockSpec((1, k), lambda i: (0, 0)),
        ],
        out_specs=[
            pl.BlockSpec((1, 1, _BR), lambda i: (i, 0, 0)),
            pl.BlockSpec(memory_space=pltpu.SMEM, block_shape=(1, 1),
                         index_map=lambda i: (0, 0)),
        ],
        out_shape=[
            jax.ShapeDtypeStruct((nblk, 1, _BR), jnp.int32),
            jax.ShapeDtypeStruct((1, 1), jnp.float32),
        ],
    )(z_part, embedding_weight, e_sq_row)


@jax.jit
def kernel(z_e_flat, embedding_weight):
    z_e_flat = z_e_flat.astype(jnp.float32)
    B, N, D = z_e_flat.shape
    K = embedding_weight.shape[0]
    M = B * N
    z_flat = z_e_flat.reshape(M, D)

    e_sq_row = jnp.sum(jnp.square(embedding_weight), axis=1)[None, :]
    idx3, loss = _vq_tc_call(z_flat, embedding_weight, e_sq_row, K, D)
    idx_flat = idx3.reshape(M)
    zq = _sc_gather_call(embedding_weight, idx_flat, M, D)

    z_q = zq.reshape(B, N, D)
    indices = idx_flat.reshape(B, N)
    commit_loss = loss[0, 0] * (0.25 / (M * D))
    return (z_q, indices, commit_loss)


# speedup vs baseline: 1.1302x; 1.1302x over previous
import functools

import jax
import jax.numpy as jnp
from jax import lax
from jax.experimental import pallas as pl
from jax.experimental.pallas import tpu as pltpu
from jax.experimental.pallas import tpu_sc as plsc

_BR = 1024


def _vq_block(z_ref, e_ref, idx_ref, loss_ref):
    i = pl.program_id(0)
    z = z_ref[...]
    e = e_ref[...]
    z_sq = jnp.sum(jnp.square(z), axis=1, keepdims=True)
    e_sq = jnp.sum(jnp.square(e), axis=1)
    m2_z_dot_e = jax.lax.dot_general(
        z * -2.0, e, (((1,), (1,)), ((), ())),
        preferred_element_type=jnp.float32)
    dist = (z_sq + m2_z_dot_e) + e_sq[None, :]
    idx = jnp.argmin(dist, axis=1).astype(jnp.int32)
    idx_ref[0, 0, :] = idx
    part = jnp.sum(jnp.min(dist, axis=1))

    @pl.when(i == 0)
    def _init():
        loss_ref[0, 0] = part

    @pl.when(i != 0)
    def _acc():
        loss_ref[0, 0] += part


def _sc_gather_call(table, idx_flat, m, d):
    info = plsc.get_sparse_core_info()
    nw = info.num_cores * info.num_subcores
    b_per_w = m // nw
    chunk = 96
    nchunk = b_per_w // chunk
    mesh = plsc.VectorSubcoreMesh(core_axis_name="c", subcore_axis_name="s")

    @functools.partial(
        pl.kernel, mesh=mesh,
        compiler_params=pltpu.CompilerParams(use_tc_tiling_on_sc=False),
        out_type=jax.ShapeDtypeStruct((m, d), jnp.float32),
        scratch_types=[
            pltpu.VMEM((b_per_w,), jnp.int32),
            pltpu.VMEM((b_per_w, d), jnp.float32),
            pltpu.SemaphoreType.DMA,
        ],
    )
    def gather(table_hbm, idx_hbm, out_hbm, idx_v, rows_v, sem):
        wid = lax.axis_index("s") * info.num_cores + lax.axis_index("c")
        base = wid * b_per_w
        pltpu.sync_copy(idx_hbm.at[pl.ds(base, b_per_w)], idx_v)
        copies = [
            pltpu.async_copy(
                table_hbm.at[idx_v.at[pl.ds(j * chunk, chunk)]],
                rows_v.at[pl.ds(j * chunk, chunk)], sem)
            for j in range(nchunk)
        ]
        for c in copies:
            c.wait()
        pltpu.sync_copy(rows_v, out_hbm.at[pl.ds(base, b_per_w)])

    return gather(table, idx_flat)


def _vq_tc_call(z_part, embedding_weight, k, d):
    m = z_part.shape[0]
    nblk = m // _BR
    return pl.pallas_call(
        _vq_block,
        grid=(nblk,),
        in_specs=[
            pl.BlockSpec((_BR, d), lambda i: (i, 0)),
            pl.BlockSpec((k, d), lambda i: (0, 0)),
        ],
        out_specs=[
            pl.BlockSpec((1, 1, _BR), lambda i: (i, 0, 0)),
            pl.BlockSpec(memory_space=pltpu.SMEM, block_shape=(1, 1),
                         index_map=lambda i: (0, 0)),
        ],
        out_shape=[
            jax.ShapeDtypeStruct((nblk, 1, _BR), jnp.int32),
            jax.ShapeDtypeStruct((1, 1), jnp.float32),
        ],
    )(z_part, embedding_weight)


@jax.jit
def kernel(z_e_flat, embedding_weight):
    z_e_flat = z_e_flat.astype(jnp.float32)
    B, N, D = z_e_flat.shape
    K = embedding_weight.shape[0]
    M = B * N
    z_flat = z_e_flat.reshape(M, D)

    idx3, loss = _vq_tc_call(z_flat, embedding_weight, K, D)
    idx_flat = idx3.reshape(M)
    zq = _sc_gather_call(embedding_weight, idx_flat, M, D)

    z_q = zq.reshape(B, N, D)
    indices = idx_flat.reshape(B, N)
    commit_loss = loss[0, 0] * (0.25 / (M * D))
    return (z_q, indices, commit_loss)
